# Initial kernel scaffold; baseline (speedup 1.0000x reference)
import jax
import jax.numpy as jnp
from jax.experimental import pallas as pl


def kernel(memory, idx, val):
    raise NotImplementedError("write your pallas kernel here")


# speedup vs baseline: 1.4758x; 1.4758x over previous
import functools

import jax
import jax.numpy as jnp
from jax import lax
from jax.experimental import pallas as pl
from jax.experimental.pallas import tpu as pltpu
from jax.experimental.pallas import tpu_sc as plsc

_NC = 2
_NS = 16
_NW = _NC * _NS
_CH = 128

_COPY_BLK = 2000


def _copy_body(x_ref, o_ref):
    o_ref[...] = x_ref[...]


@functools.cache
def _make_sc_scatter(m, d, b):
    per_w = b // _NW
    nch = per_w // _CH
    mesh = plsc.VectorSubcoreMesh(core_axis_name="c", subcore_axis_name="s")

    @functools.partial(
        pl.kernel,
        mesh=mesh,
        out_type=(),
        compiler_params=pltpu.CompilerParams(use_tc_tiling_on_sc=False),
        scratch_types=[
            pltpu.VMEM((nch, _CH), jnp.int32),
            pltpu.VMEM((nch, _CH), jnp.int32),
            pltpu.VMEM((nch, _CH, d), jnp.float32),
            pltpu.SemaphoreType.DMA,
            pltpu.SemaphoreType.DMA,
        ],
    )
    def sc_scatter(out_hbm, idx_hbm, win_hbm, val_hbm, idxb, winb, rows,
                   gsem, ssem):
        wid = lax.axis_index("s") * _NC + lax.axis_index("c")
        base = wid * per_w
        for j in range(nch):
            pltpu.sync_copy(idx_hbm.at[pl.ds(base + j * _CH, _CH)], idxb.at[j])
            pltpu.sync_copy(win_hbm.at[pl.ds(base + j * _CH, _CH)], winb.at[j])
        gathers = [
            pltpu.async_copy(val_hbm.at[winb.at[j]], rows.at[j], gsem)
            for j in range(nch)
        ]
        for g in gathers:
            g.wait()
        scatters = [
            pltpu.async_copy(rows.at[j], out_hbm.at[idxb.at[j]], ssem)
            for j in range(nch)
        ]
        for s in scatters:
            s.wait()

    return sc_scatter


def kernel(memory, idx, val):
    m, d = memory.shape
    b = idx.shape[0]

    order = jnp.argsort(idx, stable=True)
    s = idx[order]
    pos = jnp.searchsorted(s, idx, side="right") - 1
    winner = order[pos].astype(jnp.int32)

    r = (m * d) // 128
    memf = memory.reshape(r, 128)
    mem2 = pl.pallas_call(
        _copy_body,
        grid=(r // _COPY_BLK,),
        in_specs=[pl.BlockSpec((_COPY_BLK, 128), lambda i: (i, 0))],
        out_specs=pl.BlockSpec((_COPY_BLK, 128), lambda i: (i, 0)),
        out_shape=jax.ShapeDtypeStruct((r, 128), jnp.float32),
    )(memf).reshape(m, d)

    out_ref = jax.new_ref(mem2)
    _make_sc_scatter(m, d, b)(out_ref, idx, winner, val)
    return out_ref[...]
